# Initial kernel scaffold; baseline (speedup 1.0000x reference)
import functools

import jax
import jax.numpy as jnp
from jax import lax
from jax.experimental import pallas as pl
from jax.experimental.pallas import tpu as pltpu, tpu_sc as plsc

N = 10000
F = 256
NP = 10240
E0 = 160000
EP = 172032
DUMMY = 10000

NC, NS = 2, 16
NW = NC * NS
EW_A = EP // NW
CH = 128
NCH_A = EW_A // CH
EB_B = EP // NS
NCH_B = EB_B // CH
HALF = NP // 2
DROWS = NP // 16
HROWS = HALF // 16
OSH = HALF + CH


def _k1_body(x_ref, w_ref, b_ref, h_ref, hn_ref):
    xm = x_ref[...]
    acc = lax.dot_general(xm, w_ref[...], (((1,), (1,)), ((), ())),
                          preferred_element_type=jnp.float32)
    h = jnp.maximum(acc + b_ref[...], 0.0)
    h_ref[...] = h
    n = jnp.sqrt(jnp.sum(h * h, axis=1, keepdims=True))
    hn_ref[...] = h / jnp.maximum(n, 1e-12)


def _tc_h_hn(x, W1, b1):
    grid = 10
    rows = N // grid
    return pl.pallas_call(
        _k1_body,
        grid=(grid,),
        in_specs=[
            pl.BlockSpec((rows, F), lambda i: (i, 0)),
            pl.BlockSpec((F, F), lambda i: (0, 0)),
            pl.BlockSpec((1, F), lambda i: (0, 0)),
        ],
        out_specs=[
            pl.BlockSpec((rows, F), lambda i: (i, 0)),
            pl.BlockSpec((rows, F), lambda i: (i, 0)),
        ],
        out_shape=[
            jax.ShapeDtypeStruct((N, F), jnp.float32),
            jax.ShapeDtypeStruct((N, F), jnp.float32),
        ],
    )(x, W1, b1)


def _k2_body(m_ref, beta_ref, hn_ref, hnb_ref):
    m = m_ref[...]
    n = jnp.sqrt(jnp.sum(m * m, axis=1, keepdims=True))
    hn = m / jnp.maximum(n, 1e-12)
    hn_ref[...] = hn
    hnb_ref[...] = hn * beta_ref[0, 0]


def _tc_norm(m, beta):
    return pl.pallas_call(
        _k2_body,
        grid=(2,),
        in_specs=[
            pl.BlockSpec((HALF, F), lambda i: (i, 0)),
            pl.BlockSpec((1, 1), lambda i: (0, 0)),
        ],
        out_specs=[
            pl.BlockSpec((HALF, F), lambda i: (i, 0)),
            pl.BlockSpec((HALF, F), lambda i: (i, 0)),
        ],
        out_shape=[
            jax.ShapeDtypeStruct((NP, F), jnp.float32),
            jax.ShapeDtypeStruct((NP, F), jnp.float32),
        ],
    )(m, beta)


def _k3_body(m_ref, w_ref, b_ref, y_ref):
    y_ref[...] = lax.dot_general(m_ref[...], w_ref[...],
                                 (((1,), (0,)), ((), ())),
                                 preferred_element_type=jnp.float32) + b_ref[0, 0]


def _tc_out(m, W2c, b2):
    return pl.pallas_call(
        _k3_body,
        grid=(2,),
        in_specs=[
            pl.BlockSpec((HALF, F), lambda i: (i, 0)),
            pl.BlockSpec((F, 1), lambda i: (0, 0)),
            pl.BlockSpec((1, 1), lambda i: (0, 0)),
        ],
        out_specs=pl.BlockSpec((HALF, 1), lambda i: (i, 0)),
        out_shape=jax.ShapeDtypeStruct((NP, 1), jnp.float32),
    )(m, W2c, b2)


def _sc_a_body(hs_hbm, hd_hbm, src_hbm, dst_hbm, ex_hbm, dp_hbm,
               sidx, didx, rows_s, rows_d, alb, exb, dloc):
    cid = lax.axis_index("c")
    sid = lax.axis_index("s")
    wid = sid * NC + cid

    def zero_row(i, _):
        dloc[i, :] = jnp.zeros((16,), jnp.float32)
        return 0
    lax.fori_loop(0, DROWS, zero_row, 0)

    def chunk(g, _):
        base = wid * EW_A + g * CH
        pltpu.sync_copy(src_hbm.at[pl.ds(base, CH)], sidx)
        pltpu.sync_copy(dst_hbm.at[pl.ds(base, CH)], didx)
        pltpu.sync_copy(hs_hbm.at[sidx], rows_s)
        pltpu.sync_copy(hd_hbm.at[didx], rows_d)

        def dot_edge(j, _):
            acc = jnp.zeros((16,), jnp.float32)
            for k in range(16):
                acc = acc + rows_s[j, pl.ds(k * 16, 16)] * rows_d[j, pl.ds(k * 16, 16)]
            alb[j] = jnp.sum(acc)
            return 0
        lax.fori_loop(0, CH, dot_edge, 0)

        for q in range(CH // 16):
            exb[pl.ds(q * 16, 16)] = jnp.exp(alb[pl.ds(q * 16, 16)])

        def accum(j, _):
            d = didx[j]
            r = lax.shift_right_logical(d, 4)
            c = jnp.bitwise_and(d, 15)
            dloc[r, c] = dloc[r, c] + exb[j]
            return 0
        lax.fori_loop(0, CH, accum, 0)

        pltpu.sync_copy(exb, ex_hbm.at[pl.ds(base, CH)])
        return 0

    lax.fori_loop(0, NCH_A, chunk, 0)
    pltpu.sync_copy(dloc, dp_hbm.at[wid])


def _sc_alpha(hs, hd, src, dst):
    mesh = plsc.VectorSubcoreMesh(core_axis_name="c", subcore_axis_name="s",
                                  num_cores=NC, num_subcores=NS)
    return pl.kernel(
        _sc_a_body,
        out_type=[
            jax.ShapeDtypeStruct((EP,), jnp.float32),
            jax.ShapeDtypeStruct((NW, DROWS, 16), jnp.float32),
        ],
        mesh=mesh,
        scratch_types=[
            pltpu.VMEM((CH,), jnp.int32),
            pltpu.VMEM((CH,), jnp.int32),
            pltpu.VMEM((CH, F), jnp.float32),
            pltpu.VMEM((CH, F), jnp.float32),
            pltpu.VMEM((CH,), jnp.float32),
            pltpu.VMEM((CH,), jnp.float32),
            pltpu.VMEM((DROWS, 16), jnp.float32),
        ],
    )(hs, hd, src, dst)


def _sc_b_body(h_hbm, src_hbm, dst_hbm, ex_hbm, dp_hbm, out_hbm,
               sidx, didx, exv, dlv, cfv, rows, tmp, dsum, out_sh):
    cid = lax.axis_index("c")
    sid = lax.axis_index("s")
    lo = cid * HALF
    off = cid * HROWS

    pltpu.sync_copy(dp_hbm.at[0, pl.ds(off, HROWS)], dsum.at[pl.ds(0, HROWS)])

    def red_p(p, _):
        pltpu.sync_copy(dp_hbm.at[p, pl.ds(off, HROWS)], tmp)

        def red_r(i, _):
            dsum[i, :] = dsum[i, :] + tmp[i, :]
            return 0
        lax.fori_loop(0, HROWS, red_r, 0)
        return 0
    lax.fori_loop(1, NW, red_p, 0)
    dsum[HROWS, :] = jnp.ones((16,), jnp.float32)

    def zrow(i, _):
        for k in range(16):
            rows[i, pl.ds(k * 16, 16)] = jnp.zeros((16,), jnp.float32)
        return 0
    lax.fori_loop(0, CH, zrow, 0)
    zbase = sid * (OSH // NS)
    pltpu.sync_copy(rows, out_sh.at[pl.ds(zbase, CH)])
    pltpu.sync_copy(rows, out_sh.at[pl.ds(zbase + CH, CH)])
    pltpu.sync_copy(rows.at[pl.ds(0, OSH // NS - 2 * CH)],
                    out_sh.at[pl.ds(zbase + 2 * CH, OSH // NS - 2 * CH)])
    plsc.subcore_barrier()

    def chunk(g, _):
        base = sid * EB_B + g * CH
        pltpu.sync_copy(src_hbm.at[pl.ds(base, CH)], sidx)
        pltpu.sync_copy(dst_hbm.at[pl.ds(base, CH)], didx)
        pltpu.sync_copy(ex_hbm.at[pl.ds(base, CH)], exv)
        pltpu.sync_copy(h_hbm.at[sidx], rows)

        for q in range(CH // 16):
            dv = didx[pl.ds(q * 16, 16)]
            own = jnp.logical_and(dv >= lo, dv < lo + HALF)
            dl = jnp.where(own, dv - lo, HALF)
            dlv[pl.ds(q * 16, 16)] = dl
            r = lax.shift_right_logical(dl, 4)
            c = jnp.bitwise_and(dl, 15)
            dg = plsc.load_gather(dsum, [r, c])
            cf = jnp.where(own, exv[pl.ds(q * 16, 16)] / dg, 0.0)
            cfv[pl.ds(q * 16, 16)] = cf

        def scale(j, _):
            cs = cfv[j]
            for k in range(16):
                rows[j, pl.ds(k * 16, 16)] = rows[j, pl.ds(k * 16, 16)] * cs
            return 0
        lax.fori_loop(0, CH, scale, 0)

        pltpu.sync_copy(rows, out_sh.at[dlv], add=True)
        return 0
    lax.fori_loop(0, NCH_B, chunk, 0)

    plsc.subcore_barrier()
    for o, n in ((0, CH), (CH, CH), (2 * CH, HROWS - 2 * CH)):
        b = sid * HROWS + o
        pltpu.sync_copy(out_sh.at[pl.ds(b, n)], rows.at[pl.ds(0, n)])
        pltpu.sync_copy(rows.at[pl.ds(0, n)], out_hbm.at[pl.ds(lo + b, n)])


def _sc_scatter(h, src, dst, ex, dp):
    mesh = plsc.VectorSubcoreMesh(core_axis_name="c", subcore_axis_name="s",
                                  num_cores=NC, num_subcores=NS)
    return pl.kernel(
        _sc_b_body,
        out_type=jax.ShapeDtypeStruct((NP, F), jnp.float32),
        mesh=mesh,
        scratch_types=[
            pltpu.VMEM((CH,), jnp.int32),
            pltpu.VMEM((CH,), jnp.int32),
            pltpu.VMEM((CH,), jnp.float32),
            pltpu.VMEM((CH,), jnp.int32),
            pltpu.VMEM((CH,), jnp.float32),
            pltpu.VMEM((CH, F), jnp.float32),
            pltpu.VMEM((HROWS, 16), jnp.float32),
            pltpu.VMEM((HROWS + 8, 16), jnp.float32),
            pltpu.VMEM_SHARED((OSH, F), jnp.float32),
        ],
    )(h, src, dst, ex, dp)


@jax.jit
def kernel(x, edge_index, W1, b1, beta2, W2, b2):
    src = edge_index[0].astype(jnp.int32)
    dst = edge_index[1].astype(jnp.int32)
    loop = jnp.arange(N, dtype=jnp.int32)
    padi = jnp.full((EP - E0 - N,), DUMMY, jnp.int32)
    src = jnp.concatenate([src, loop, padi])
    dst = jnp.concatenate([dst, loop, padi])

    h, hn = _tc_h_hn(x, W1, b1.reshape(1, F))
    hp = jnp.zeros((NP, F), jnp.float32).at[:N].set(h)
    hnp = jnp.zeros((NP, F), jnp.float32).at[:N].set(hn)

    ex1, dp1 = _sc_alpha(hnp, hnp, src, dst)
    out1 = _sc_scatter(hp, src, dst, ex1, dp1)

    hn2, hn2b = _tc_norm(out1, beta2.reshape(1, 1))
    ex2, dp2 = _sc_alpha(hn2b, hn2, src, dst)
    out2 = _sc_scatter(out1, src, dst, ex2, dp2)

    y = _tc_out(out2, W2.reshape(F, 1), b2.reshape(1, 1))
    return (y.reshape(-1)[:N],)


# speedup vs baseline: 3.5849x; 3.5849x over previous
import jax
import jax.numpy as jnp
from jax import lax
from jax.experimental import pallas as pl
from jax.experimental.pallas import tpu as pltpu, tpu_sc as plsc

N = 10000
F = 256
NP = 10240
E0 = 160000
EP = 172032
DUMMY = 10000

NC, NS = 2, 16
NW = NC * NS
EW_A = EP // NW
CH = 128
NCH_A = EW_A // CH
EB_B = EP // NS
NCH_B = EB_B // CH
HALF = NP // 2
DROWS = NP // 16
HROWS = HALF // 16
OSH = HALF + CH


def _k1_body(x_ref, w_ref, b_ref, h_ref, hn_ref):
    xm = x_ref[...]
    acc = lax.dot_general(xm, w_ref[...], (((1,), (1,)), ((), ())),
                          preferred_element_type=jnp.float32)
    h = jnp.maximum(acc + b_ref[...], 0.0)
    h_ref[...] = h
    n = jnp.sqrt(jnp.sum(h * h, axis=1, keepdims=True))
    hn_ref[...] = h / jnp.maximum(n, 1e-12)


def _tc_h_hn(x, W1, b1):
    grid = 10
    rows = N // grid
    return pl.pallas_call(
        _k1_body,
        grid=(grid,),
        in_specs=[
            pl.BlockSpec((rows, F), lambda i: (i, 0)),
            pl.BlockSpec((F, F), lambda i: (0, 0)),
            pl.BlockSpec((1, F), lambda i: (0, 0)),
        ],
        out_specs=[
            pl.BlockSpec((rows, F), lambda i: (i, 0)),
            pl.BlockSpec((rows, F), lambda i: (i, 0)),
        ],
        out_shape=[
            jax.ShapeDtypeStruct((N, F), jnp.float32),
            jax.ShapeDtypeStruct((N, F), jnp.float32),
        ],
    )(x, W1, b1)


def _k2_body(m_ref, beta_ref, hn_ref, hnb_ref):
    m = m_ref[...]
    n = jnp.sqrt(jnp.sum(m * m, axis=1, keepdims=True))
    hn = m / jnp.maximum(n, 1e-12)
    hn_ref[...] = hn
    hnb_ref[...] = hn * beta_ref[0, 0]


def _tc_norm(m, beta):
    return pl.pallas_call(
        _k2_body,
        grid=(2,),
        in_specs=[
            pl.BlockSpec((HALF, F), lambda i: (i, 0)),
            pl.BlockSpec((1, 1), lambda i: (0, 0)),
        ],
        out_specs=[
            pl.BlockSpec((HALF, F), lambda i: (i, 0)),
            pl.BlockSpec((HALF, F), lambda i: (i, 0)),
        ],
        out_shape=[
            jax.ShapeDtypeStruct((NP, F), jnp.float32),
            jax.ShapeDtypeStruct((NP, F), jnp.float32),
        ],
    )(m, beta)


def _k3_body(m_ref, w_ref, b_ref, y_ref):
    y_ref[...] = lax.dot_general(m_ref[...], w_ref[...],
                                 (((1,), (0,)), ((), ())),
                                 preferred_element_type=jnp.float32) + b_ref[0, 0]


def _tc_out(m, W2c, b2):
    return pl.pallas_call(
        _k3_body,
        grid=(2,),
        in_specs=[
            pl.BlockSpec((HALF, F), lambda i: (i, 0)),
            pl.BlockSpec((F, 1), lambda i: (0, 0)),
            pl.BlockSpec((1, 1), lambda i: (0, 0)),
        ],
        out_specs=pl.BlockSpec((HALF, 1), lambda i: (i, 0)),
        out_shape=jax.ShapeDtypeStruct((NP, 1), jnp.float32),
    )(m, W2c, b2)


def _sc_a_body(hs_hbm, hd_hbm, src_hbm, dst_hbm, ex_hbm, dp_hbm,
               sidx, didx, rows_s, rows_d, exb, dloc):
    cid = lax.axis_index("c")
    sid = lax.axis_index("s")
    wid = sid * NC + cid
    lanes = lax.broadcasted_iota(jnp.int32, (16,), 0)

    def zero_row(i, _):
        dloc[i, :] = jnp.zeros((16,), jnp.float32)
        return 0
    lax.fori_loop(0, DROWS, zero_row, 0)

    def chunk(g, _):
        base = wid * EW_A + g * CH
        pltpu.sync_copy(src_hbm.at[pl.ds(base, CH)], sidx)
        pltpu.sync_copy(dst_hbm.at[pl.ds(base, CH)], didx)
        pltpu.sync_copy(hs_hbm.at[sidx], rows_s)
        pltpu.sync_copy(hd_hbm.at[didx], rows_d)

        def group(q, _):
            def dot_edge(j2, av):
                j = q * 16 + j2
                acc = jnp.zeros((16,), jnp.float32)
                for k in range(16):
                    acc = acc + rows_s[j, pl.ds(k * 16, 16)] * rows_d[j, pl.ds(k * 16, 16)]
                for sh in (8, 4, 2, 1):
                    acc = acc + acc.at[lanes ^ sh].get(mode="promise_in_bounds")
                return jnp.where(lanes == j2, acc, av)
            av = lax.fori_loop(0, 16, dot_edge, jnp.zeros((16,), jnp.float32))
            ev = jnp.exp(av)
            exb[pl.ds(q * 16, 16)] = ev
            dvec = didx[pl.ds(q * 16, 16)]
            rvec = jnp.right_shift(dvec, 4)
            cvec = jnp.bitwise_and(dvec, 15)
            for j2 in range(16):
                onehot = jnp.where(lanes == cvec[j2], ev[j2], 0.0)
                dloc[rvec[j2], :] = dloc[rvec[j2], :] + onehot
            return 0
        lax.fori_loop(0, CH // 16, group, 0)

        pltpu.sync_copy(exb, ex_hbm.at[pl.ds(base, CH)])
        return 0

    lax.fori_loop(0, NCH_A, chunk, 0)
    pltpu.sync_copy(dloc, dp_hbm.at[wid])


def _sc_alpha(hs, hd, src, dst):
    mesh = plsc.VectorSubcoreMesh(core_axis_name="c", subcore_axis_name="s",
                                  num_cores=NC, num_subcores=NS)
    return pl.kernel(
        _sc_a_body,
        out_type=[
            jax.ShapeDtypeStruct((EP,), jnp.float32),
            jax.ShapeDtypeStruct((NW, DROWS, 16), jnp.float32),
        ],
        mesh=mesh,
        compiler_params=pltpu.CompilerParams(use_tc_tiling_on_sc=False),
        scratch_types=[
            pltpu.VMEM((CH,), jnp.int32),
            pltpu.VMEM((CH,), jnp.int32),
            pltpu.VMEM((CH, F), jnp.float32),
            pltpu.VMEM((CH, F), jnp.float32),
            pltpu.VMEM((CH,), jnp.float32),
            pltpu.VMEM((DROWS, 16), jnp.float32),
        ],
    )(hs, hd, src, dst)


def _sc_b_body(h_hbm, src_hbm, dst_hbm, ex_hbm, dp_hbm, out_hbm,
               sidx, didx, exv, dlv, rows, tmp, dsum, out_sh):
    cid = lax.axis_index("c")
    sid = lax.axis_index("s")
    lo = cid * HALF
    off = cid * HROWS
    lanes = lax.broadcasted_iota(jnp.int32, (16,), 0)

    pltpu.sync_copy(dp_hbm.at[0, pl.ds(off, HROWS)], dsum.at[pl.ds(0, HROWS)])

    def red_p(p, _):
        pltpu.sync_copy(dp_hbm.at[p, pl.ds(off, HROWS)], tmp)

        def red_r(i, _):
            dsum[i, :] = dsum[i, :] + tmp[i, :]
            return 0
        lax.fori_loop(0, HROWS, red_r, 0)
        return 0
    lax.fori_loop(1, NW, red_p, 0)
    dsum[HROWS, :] = jnp.ones((16,), jnp.float32)

    def zrow(i, _):
        for k in range(16):
            rows[i, pl.ds(k * 16, 16)] = jnp.zeros((16,), jnp.float32)
        return 0
    lax.fori_loop(0, CH, zrow, 0)
    zbase = sid * (OSH // NS)
    pltpu.sync_copy(rows, out_sh.at[pl.ds(zbase, CH)])
    pltpu.sync_copy(rows, out_sh.at[pl.ds(zbase + CH, CH)])
    pltpu.sync_copy(rows.at[pl.ds(0, OSH // NS - 2 * CH)],
                    out_sh.at[pl.ds(zbase + 2 * CH, OSH // NS - 2 * CH)])
    plsc.subcore_barrier()

    def chunk(g, _):
        base = sid * EB_B + g * CH
        pltpu.sync_copy(src_hbm.at[pl.ds(base, CH)], sidx)
        pltpu.sync_copy(dst_hbm.at[pl.ds(base, CH)], didx)
        pltpu.sync_copy(ex_hbm.at[pl.ds(base, CH)], exv)
        pltpu.sync_copy(h_hbm.at[sidx], rows)

        def group(q, _):
            dv = didx[pl.ds(q * 16, 16)]
            own = jnp.logical_and(dv >= lo, dv < lo + HALF)
            dl = jnp.where(own, dv - lo, HALF)
            dlv[pl.ds(q * 16, 16)] = dl
            rvec = jnp.right_shift(dl, 4)
            cvec = jnp.bitwise_and(dl, 15)
            dg = jnp.zeros((16,), jnp.float32)
            for j2 in range(16):
                row = dsum[rvec[j2], :]
                rowsel = row.at[cvec].get(mode="promise_in_bounds")
                dg = jnp.where(lanes == j2, rowsel, dg)
            cf = jnp.where(own, exv[pl.ds(q * 16, 16)] / dg, 0.0)
            for j2 in range(16):
                j = q * 16 + j2
                cs = cf[j2]
                for k in range(16):
                    rows[j, pl.ds(k * 16, 16)] = rows[j, pl.ds(k * 16, 16)] * cs
            return 0
        lax.fori_loop(0, CH // 16, group, 0)

        pltpu.sync_copy(rows, out_sh.at[dlv], add=True)
        return 0
    lax.fori_loop(0, NCH_B, chunk, 0)

    plsc.subcore_barrier()
    for o, n in ((0, CH), (CH, CH), (2 * CH, HROWS - 2 * CH)):
        b = sid * HROWS + o
        pltpu.sync_copy(out_sh.at[pl.ds(b, n)], rows.at[pl.ds(0, n)])
        pltpu.sync_copy(rows.at[pl.ds(0, n)], out_hbm.at[pl.ds(lo + b, n)])


def _sc_scatter(h, src, dst, ex, dp):
    mesh = plsc.VectorSubcoreMesh(core_axis_name="c", subcore_axis_name="s",
                                  num_cores=NC, num_subcores=NS)
    return pl.kernel(
        _sc_b_body,
        out_type=jax.ShapeDtypeStruct((NP, F), jnp.float32),
        mesh=mesh,
        compiler_params=pltpu.CompilerParams(use_tc_tiling_on_sc=False),
        scratch_types=[
            pltpu.VMEM((CH,), jnp.int32),
            pltpu.VMEM((CH,), jnp.int32),
            pltpu.VMEM((CH,), jnp.float32),
            pltpu.VMEM((CH,), jnp.int32),
            pltpu.VMEM((CH, F), jnp.float32),
            pltpu.VMEM((HROWS, 16), jnp.float32),
            pltpu.VMEM((HROWS + 8, 16), jnp.float32),
            pltpu.VMEM_SHARED((OSH, F), jnp.float32),
        ],
    )(h, src, dst, ex, dp)


@jax.jit
def kernel(x, edge_index, W1, b1, beta2, W2, b2):
    src = edge_index[0].astype(jnp.int32)
    dst = edge_index[1].astype(jnp.int32)
    loop = jnp.arange(N, dtype=jnp.int32)
    padi = jnp.full((EP - E0 - N,), DUMMY, jnp.int32)
    src = jnp.concatenate([src, loop, padi])
    dst = jnp.concatenate([dst, loop, padi])

    h, hn = _tc_h_hn(x, W1, b1.reshape(1, F))
    hp = jnp.zeros((NP, F), jnp.float32).at[:N].set(h)
    hnp = jnp.zeros((NP, F), jnp.float32).at[:N].set(hn)

    ex1, dp1 = _sc_alpha(hnp, hnp, src, dst)
    out1 = _sc_scatter(hp, src, dst, ex1, dp1)

    hn2, hn2b = _tc_norm(out1, beta2.reshape(1, 1))
    ex2, dp2 = _sc_alpha(hn2b, hn2, src, dst)
    out2 = _sc_scatter(out1, src, dst, ex2, dp2)

    y = _tc_out(out2, W2.reshape(F, 1), b2.reshape(1, 1))
    return (y.reshape(-1)[:N],)


# speedup vs baseline: 4.2370x; 1.1819x over previous
import jax
import jax.numpy as jnp
from jax import lax
from jax.experimental import pallas as pl
from jax.experimental.pallas import tpu as pltpu, tpu_sc as plsc

N = 10000
F = 256
NP = 10240
E0 = 160000
EP = 172032
DUMMY = 10000

NC, NS = 2, 16
NW = NC * NS
EW_A = EP // NW
CH = 128
NCH_A = EW_A // CH
EB_B = EP // NS
NCH_B = EB_B // CH
HALF = NP // 2
DROWS = NP // 16
HROWS = HALF // 16
OSH = HALF + CH


def _k1_body(x_ref, w_ref, b_ref, h_ref, hn_ref):
    xm = x_ref[...]
    acc = lax.dot_general(xm, w_ref[...], (((1,), (1,)), ((), ())),
                          preferred_element_type=jnp.float32)
    h = jnp.maximum(acc + b_ref[...], 0.0)
    h_ref[...] = h
    n = jnp.sqrt(jnp.sum(h * h, axis=1, keepdims=True))
    hn_ref[...] = h / jnp.maximum(n, 1e-12)


def _tc_h_hn(x, W1, b1):
    grid = 10
    rows = N // grid
    return pl.pallas_call(
        _k1_body,
        grid=(grid,),
        in_specs=[
            pl.BlockSpec((rows, F), lambda i: (i, 0)),
            pl.BlockSpec((F, F), lambda i: (0, 0)),
            pl.BlockSpec((1, F), lambda i: (0, 0)),
        ],
        out_specs=[
            pl.BlockSpec((rows, F), lambda i: (i, 0)),
            pl.BlockSpec((rows, F), lambda i: (i, 0)),
        ],
        out_shape=[
            jax.ShapeDtypeStruct((N, F), jnp.float32),
            jax.ShapeDtypeStruct((N, F), jnp.float32),
        ],
    )(x, W1, b1)


def _k2_body(m_ref, beta_ref, hn_ref, hnb_ref):
    m = m_ref[...]
    n = jnp.sqrt(jnp.sum(m * m, axis=1, keepdims=True))
    hn = m / jnp.maximum(n, 1e-12)
    hn_ref[...] = hn
    hnb_ref[...] = hn * beta_ref[0, 0]


def _tc_norm(m, beta):
    return pl.pallas_call(
        _k2_body,
        grid=(2,),
        in_specs=[
            pl.BlockSpec((HALF, F), lambda i: (i, 0)),
            pl.BlockSpec((1, 1), lambda i: (0, 0)),
        ],
        out_specs=[
            pl.BlockSpec((HALF, F), lambda i: (i, 0)),
            pl.BlockSpec((HALF, F), lambda i: (i, 0)),
        ],
        out_shape=[
            jax.ShapeDtypeStruct((NP, F), jnp.float32),
            jax.ShapeDtypeStruct((NP, F), jnp.float32),
        ],
    )(m, beta)


def _k3_body(m_ref, w_ref, b_ref, y_ref):
    y_ref[...] = lax.dot_general(m_ref[...], w_ref[...],
                                 (((1,), (0,)), ((), ())),
                                 preferred_element_type=jnp.float32) + b_ref[0, 0]


def _tc_out(m, W2c, b2):
    return pl.pallas_call(
        _k3_body,
        grid=(2,),
        in_specs=[
            pl.BlockSpec((HALF, F), lambda i: (i, 0)),
            pl.BlockSpec((F, 1), lambda i: (0, 0)),
            pl.BlockSpec((1, 1), lambda i: (0, 0)),
        ],
        out_specs=pl.BlockSpec((HALF, 1), lambda i: (i, 0)),
        out_shape=jax.ShapeDtypeStruct((NP, 1), jnp.float32),
    )(m, W2c, b2)


CHA = 64
NCHA = EW_A // CHA


def _sc_a_body(hs_hbm, hd_hbm, src_hbm, dst_hbm, ex_hbm, dp_hbm,
               sidx_all, didx_all, rs2, rd2, exb, dloc, sem0, sem1):
    cid = lax.axis_index("c")
    sid = lax.axis_index("s")
    wid = sid * NC + cid
    tb = wid * EW_A
    lanes = lax.broadcasted_iota(jnp.int32, (16,), 0)
    sems = (sem0, sem1)

    pltpu.sync_copy(src_hbm.at[pl.ds(tb, EW_A)], sidx_all)
    pltpu.sync_copy(dst_hbm.at[pl.ds(tb, EW_A)], didx_all)

    def zero_row(i, _):
        dloc[i, :] = jnp.zeros((16,), jnp.float32)
        return 0
    lax.fori_loop(0, DROWS, zero_row, 0)

    def issue(gc, b):
        pltpu.async_copy(hs_hbm.at[sidx_all.at[pl.ds(gc * CHA, CHA)]],
                         rs2.at[b], sems[b])
        pltpu.async_copy(hd_hbm.at[didx_all.at[pl.ds(gc * CHA, CHA)]],
                         rd2.at[b], sems[b])

    def wait(gc, b):
        pltpu.make_async_copy(hs_hbm.at[sidx_all.at[pl.ds(gc * CHA, CHA)]],
                              rs2.at[b], sems[b]).wait()
        pltpu.make_async_copy(hd_hbm.at[didx_all.at[pl.ds(gc * CHA, CHA)]],
                              rd2.at[b], sems[b]).wait()

    def compute(g, b):
        rows_s = rs2.at[b]
        rows_d = rd2.at[b]

        def group(q, _):
            def dot_edge(j2, av):
                j = q * 16 + j2
                acc = jnp.zeros((16,), jnp.float32)
                for k in range(16):
                    acc = acc + rows_s[j, pl.ds(k * 16, 16)] * rows_d[j, pl.ds(k * 16, 16)]
                for sh in (8, 4, 2, 1):
                    acc = acc + acc.at[lanes ^ sh].get(mode="promise_in_bounds")
                return jnp.where(lanes == j2, acc, av)
            av = lax.fori_loop(0, 16, dot_edge, jnp.zeros((16,), jnp.float32))
            ev = jnp.exp(av)
            exb[pl.ds(q * 16, 16)] = ev
            dvec = didx_all[pl.ds(g * CHA + q * 16, 16)]
            rvec = jnp.right_shift(dvec, 4)
            cvec = jnp.bitwise_and(dvec, 15)
            for j2 in range(16):
                onehot = jnp.where(lanes == cvec[j2], ev[j2], 0.0)
                dloc[rvec[j2], :] = dloc[rvec[j2], :] + onehot
            return 0
        lax.fori_loop(0, CHA // 16, group, 0)
        pltpu.sync_copy(exb, ex_hbm.at[pl.ds(tb + g * CHA, CHA)])

    issue(0, 0)

    def outer(p, _):
        for b in (0, 1):
            g = 2 * p + b
            issue(jnp.minimum(g + 1, NCHA - 1), 1 - b)
            wait(g, b)
            compute(g, b)
        return 0
    lax.fori_loop(0, NCHA // 2, outer, 0)
    wait(NCHA - 1, 0)

    pltpu.sync_copy(dloc, dp_hbm.at[wid])


def _sc_alpha(hs, hd, src, dst):
    mesh = plsc.VectorSubcoreMesh(core_axis_name="c", subcore_axis_name="s",
                                  num_cores=NC, num_subcores=NS)
    return pl.kernel(
        _sc_a_body,
        out_type=[
            jax.ShapeDtypeStruct((EP,), jnp.float32),
            jax.ShapeDtypeStruct((NW, DROWS, 16), jnp.float32),
        ],
        mesh=mesh,
        compiler_params=pltpu.CompilerParams(use_tc_tiling_on_sc=False),
        scratch_types=[
            pltpu.VMEM((EW_A,), jnp.int32),
            pltpu.VMEM((EW_A,), jnp.int32),
            pltpu.VMEM((2, CHA, F), jnp.float32),
            pltpu.VMEM((2, CHA, F), jnp.float32),
            pltpu.VMEM((CHA,), jnp.float32),
            pltpu.VMEM((DROWS, 16), jnp.float32),
            pltpu.SemaphoreType.DMA,
            pltpu.SemaphoreType.DMA,
        ],
    )(hs, hd, src, dst)


def _sc_b_body(h_hbm, src_hbm, dst_hbm, ex_hbm, dp_hbm, out_hbm,
               sidx, didx, exv, dlv, rows, tmp, dsum, out_sh):
    cid = lax.axis_index("c")
    sid = lax.axis_index("s")
    lo = cid * HALF
    off = cid * HROWS
    lanes = lax.broadcasted_iota(jnp.int32, (16,), 0)

    pltpu.sync_copy(dp_hbm.at[0, pl.ds(off, HROWS)], dsum.at[pl.ds(0, HROWS)])

    def red_p(p, _):
        pltpu.sync_copy(dp_hbm.at[p, pl.ds(off, HROWS)], tmp)

        def red_r(i, _):
            dsum[i, :] = dsum[i, :] + tmp[i, :]
            return 0
        lax.fori_loop(0, HROWS, red_r, 0)
        return 0
    lax.fori_loop(1, NW, red_p, 0)
    dsum[HROWS, :] = jnp.ones((16,), jnp.float32)

    def zrow(i, _):
        for k in range(16):
            rows[i, pl.ds(k * 16, 16)] = jnp.zeros((16,), jnp.float32)
        return 0
    lax.fori_loop(0, CH, zrow, 0)
    zbase = sid * (OSH // NS)
    pltpu.sync_copy(rows, out_sh.at[pl.ds(zbase, CH)])
    pltpu.sync_copy(rows, out_sh.at[pl.ds(zbase + CH, CH)])
    pltpu.sync_copy(rows.at[pl.ds(0, OSH // NS - 2 * CH)],
                    out_sh.at[pl.ds(zbase + 2 * CH, OSH // NS - 2 * CH)])
    plsc.subcore_barrier()

    def chunk(g, _):
        base = sid * EB_B + g * CH
        pltpu.sync_copy(src_hbm.at[pl.ds(base, CH)], sidx)
        pltpu.sync_copy(dst_hbm.at[pl.ds(base, CH)], didx)
        pltpu.sync_copy(ex_hbm.at[pl.ds(base, CH)], exv)
        pltpu.sync_copy(h_hbm.at[sidx], rows)

        def group(q, _):
            dv = didx[pl.ds(q * 16, 16)]
            own = jnp.logical_and(dv >= lo, dv < lo + HALF)
            dl = jnp.where(own, dv - lo, HALF)
            dlv[pl.ds(q * 16, 16)] = dl
            rvec = jnp.right_shift(dl, 4)
            cvec = jnp.bitwise_and(dl, 15)
            dg = jnp.zeros((16,), jnp.float32)
            for j2 in range(16):
                row = dsum[rvec[j2], :]
                rowsel = row.at[cvec].get(mode="promise_in_bounds")
                dg = jnp.where(lanes == j2, rowsel, dg)
            cf = jnp.where(own, exv[pl.ds(q * 16, 16)] / dg, 0.0)
            for j2 in range(16):
                j = q * 16 + j2
                cs = cf[j2]
                for k in range(16):
                    rows[j, pl.ds(k * 16, 16)] = rows[j, pl.ds(k * 16, 16)] * cs
            return 0
        lax.fori_loop(0, CH // 16, group, 0)

        pltpu.sync_copy(rows, out_sh.at[dlv], add=True)
        return 0
    lax.fori_loop(0, NCH_B, chunk, 0)

    plsc.subcore_barrier()
    for o, n in ((0, CH), (CH, CH), (2 * CH, HROWS - 2 * CH)):
        b = sid * HROWS + o
        pltpu.sync_copy(out_sh.at[pl.ds(b, n)], rows.at[pl.ds(0, n)])
        pltpu.sync_copy(rows.at[pl.ds(0, n)], out_hbm.at[pl.ds(lo + b, n)])


def _sc_scatter(h, src, dst, ex, dp):
    mesh = plsc.VectorSubcoreMesh(core_axis_name="c", subcore_axis_name="s",
                                  num_cores=NC, num_subcores=NS)
    return pl.kernel(
        _sc_b_body,
        out_type=jax.ShapeDtypeStruct((NP, F), jnp.float32),
        mesh=mesh,
        compiler_params=pltpu.CompilerParams(use_tc_tiling_on_sc=False),
        scratch_types=[
            pltpu.VMEM((CH,), jnp.int32),
            pltpu.VMEM((CH,), jnp.int32),
            pltpu.VMEM((CH,), jnp.float32),
            pltpu.VMEM((CH,), jnp.int32),
            pltpu.VMEM((CH, F), jnp.float32),
            pltpu.VMEM((HROWS, 16), jnp.float32),
            pltpu.VMEM((HROWS + 8, 16), jnp.float32),
            pltpu.VMEM_SHARED((OSH, F), jnp.float32),
        ],
    )(h, src, dst, ex, dp)


@jax.jit
def kernel(x, edge_index, W1, b1, beta2, W2, b2):
    src = edge_index[0].astype(jnp.int32)
    dst = edge_index[1].astype(jnp.int32)
    loop = jnp.arange(N, dtype=jnp.int32)
    padi = jnp.full((EP - E0 - N,), DUMMY, jnp.int32)
    src = jnp.concatenate([src, loop, padi])
    dst = jnp.concatenate([dst, loop, padi])

    h, hn = _tc_h_hn(x, W1, b1.reshape(1, F))
    hp = jnp.zeros((NP, F), jnp.float32).at[:N].set(h)
    hnp = jnp.zeros((NP, F), jnp.float32).at[:N].set(hn)

    ex1, dp1 = _sc_alpha(hnp, hnp, src, dst)
    out1 = _sc_scatter(hp, src, dst, ex1, dp1)

    hn2, hn2b = _tc_norm(out1, beta2.reshape(1, 1))
    ex2, dp2 = _sc_alpha(hn2b, hn2, src, dst)
    out2 = _sc_scatter(out1, src, dst, ex2, dp2)

    y = _tc_out(out2, W2.reshape(F, 1), b2.reshape(1, 1))
    return (y.reshape(-1)[:N],)
